# Initial kernel scaffold; baseline (speedup 1.0000x reference)
import jax
import jax.numpy as jnp
from jax.experimental import pallas as pl


def kernel(x, edge_index, batch, norm1_weight, norm1_bias, norm1_mean_scale, W_l, b_l, W_r, norm2_weight, norm2_bias, norm2_mean_scale):
    raise NotImplementedError("write your pallas kernel here")


# speedup vs baseline: 1.0073x; 1.0073x over previous
import functools

import jax
import jax.numpy as jnp
from jax import lax
from jax.experimental import pallas as pl
from jax.experimental.pallas import tpu as pltpu
from jax.experimental.pallas import tpu_sc as plsc

N = 10000
E = 320000
D = 128
G = 8
EPS = 1e-5

NC = 2
NS = 16
NW = NC * NS
NODES_PER_W = 313
NPAD = NW * NODES_PER_W
CHUNK = 8000
NCHUNK = E // CHUNK
GROUPS = CHUNK // 16
KB = 128


def _graph_norm_body(x, onehot, counts, weight, bias, mean_scale):
    sums = lax.dot_general(onehot, x, (((0,), (0,)), ((), ())),
                           preferred_element_type=jnp.float32, precision=lax.Precision.HIGHEST)
    mean = sums / counts
    meanx = lax.dot_general(onehot, mean, (((1,), (0,)), ((), ())),
                            preferred_element_type=jnp.float32, precision=lax.Precision.HIGHEST)
    out = x - meanx * mean_scale
    var = lax.dot_general(onehot, out * out, (((0,), (0,)), ((), ())),
                          preferred_element_type=jnp.float32, precision=lax.Precision.HIGHEST) / counts
    stdg = jnp.sqrt(var + EPS)
    stdx = lax.dot_general(onehot, stdg, (((1,), (0,)), ((), ())),
                           preferred_element_type=jnp.float32, precision=lax.Precision.HIGHEST)
    return weight * out / stdx + bias


def _tc_head_body(x_ref, batch_ref, w_ref, b_ref, ms_ref, wr_ref,
                  h1_ref, hwr_ref):
    x = x_ref[...]
    bt = batch_ref[...]
    onehot = (bt == lax.broadcasted_iota(jnp.int32, (1, G), 1)
              ).astype(jnp.float32)
    ones = jnp.ones((N, 1), jnp.float32)
    counts = jnp.maximum(
        lax.dot_general(onehot, ones, (((0,), (0,)), ((), ())),
                        preferred_element_type=jnp.float32, precision=lax.Precision.HIGHEST), 1.0)
    h1 = _graph_norm_body(x, onehot, counts, w_ref[...], b_ref[...],
                          ms_ref[...])
    h1_ref[...] = h1
    hwr_ref[...] = lax.dot_general(h1, wr_ref[...], (((1,), (1,)), ((), ())),
                                   preferred_element_type=jnp.float32, precision=lax.Precision.HIGHEST)


def _tc_tail_body(x_ref, agg_ref, hwr_ref, batch_ref, wl_ref, bl_ref,
                  w_ref, b_ref, ms_ref, out_ref):
    x = x_ref[...]
    t = lax.dot_general(agg_ref[...], wl_ref[...], (((1,), (1,)), ((), ())),
                        preferred_element_type=jnp.float32, precision=lax.Precision.HIGHEST)
    h2 = jnp.maximum(x + t + bl_ref[...] + hwr_ref[...], 0.0)
    bt = batch_ref[...]
    onehot = (bt == lax.broadcasted_iota(jnp.int32, (1, G), 1)
              ).astype(jnp.float32)
    ones = jnp.ones((N, 1), jnp.float32)
    counts = jnp.maximum(
        lax.dot_general(onehot, ones, (((0,), (0,)), ((), ())),
                        preferred_element_type=jnp.float32, precision=lax.Precision.HIGHEST), 1.0)
    out_ref[...] = _graph_norm_body(h2, onehot, counts, w_ref[...],
                                    b_ref[...], ms_ref[...])


_tc_head = pl.pallas_call(
    _tc_head_body,
    out_shape=(jax.ShapeDtypeStruct((N, D), jnp.float32),
               jax.ShapeDtypeStruct((N, D), jnp.float32)),
)

_tc_tail = pl.pallas_call(
    _tc_tail_body,
    out_shape=jax.ShapeDtypeStruct((N, D), jnp.float32),
)


def _sc_segmax_body(h_hbm, src_hbm, dst_hbm, out_hbm,
                    dst_buf, src_buf, gat_idx, loc_idx, rows_v, agg_v, sem):
    wid = lax.axis_index("s") * NC + lax.axis_index("c")
    lo = wid * NODES_PER_W
    iota = lax.iota(jnp.int32, 16)
    ninf = jnp.full((16,), -jnp.inf, jnp.float32)

    def init_body(i, _):
        idxv = i * 16 + iota
        plsc.store_scatter(agg_v, [idxv], ninf)
        return 0
    lax.fori_loop(0, NODES_PER_W * D // 16, init_body, 0)

    def init_idx(i, _):
        idxv = i * 16 + iota
        plsc.store_scatter(gat_idx, [idxv], jnp.zeros((16,), jnp.int32))
        return 0
    lax.fori_loop(0, (CHUNK + 16) // 16, init_idx, 0)

    lov = jnp.full((16,), lo, jnp.int32)
    hiv = lov + NODES_PER_W

    def chunk_body(c, _):
        pltpu.sync_copy(dst_hbm.at[pl.ds(c * CHUNK, CHUNK)], dst_buf)
        pltpu.sync_copy(src_hbm.at[pl.ds(c * CHUNK, CHUNK)], src_buf)

        def scan_body(g, n_vec):
            ev = g * 16 + iota
            d16 = plsc.load_gather(dst_buf, [ev])
            s16 = plsc.load_gather(src_buf, [ev])
            m = (d16 >= lov) & (d16 < hiv)
            pref = plsc.cumsum(jnp.where(m, 1, 0))
            pos = n_vec + pref - 1
            plsc.store_scatter(gat_idx, [pos], s16, mask=m)
            plsc.store_scatter(loc_idx, [pos], d16 - lov, mask=m)
            return n_vec + plsc.all_reduce_population_count(m)

        n_vec = lax.fori_loop(0, GROUPS, scan_body,
                              jnp.zeros((16,), jnp.int32))
        n = jnp.max(n_vec)

        nb = (n + KB - 1) // KB

        def batch_body(b, _):
            pltpu.async_copy(h_hbm.at[gat_idx.at[pl.ds(b * KB, KB)]],
                             rows_v, sem).wait()
            k = jnp.minimum(n - b * KB, KB)

            def edge_body(j, _):
                dv = plsc.load_gather(loc_idx, [jnp.full((16,), b * KB + j,
                                                         jnp.int32)])
                base = dv * D
                rj = jnp.full((16,), j, jnp.int32)
                for c8 in range(D // 16):
                    colv = iota + c8 * 16
                    r = plsc.load_gather(rows_v, [rj, colv])
                    a = plsc.load_gather(agg_v, [base + colv])
                    plsc.store_scatter(agg_v, [base + colv],
                                       jnp.maximum(a, r))
                return 0

            lax.fori_loop(0, k, edge_body, 0)
            return 0

        lax.fori_loop(0, nb, batch_body, 0)
        return 0

    lax.fori_loop(0, NCHUNK, chunk_body, 0)

    def fix_body(i, _):
        idxv = i * 16 + iota
        v = plsc.load_gather(agg_v, [idxv])
        v = jnp.where(v == ninf, 0.0, v)
        plsc.store_scatter(agg_v, [idxv], v)
        return 0
    lax.fori_loop(0, NODES_PER_W * D // 16, fix_body, 0)

    pltpu.sync_copy(agg_v, out_hbm.at[pl.ds(lo * D, NODES_PER_W * D)])


@functools.cache
def _sc_segmax():
  return pl.kernel(
    _sc_segmax_body,
    out_type=jax.ShapeDtypeStruct((NPAD * D,), jnp.float32),
    mesh=plsc.VectorSubcoreMesh(core_axis_name="c", subcore_axis_name="s",
                                num_cores=NC, num_subcores=NS),
    compiler_params=pltpu.CompilerParams(needs_layout_passes=False),
    scratch_types=[
        pltpu.VMEM((CHUNK,), jnp.int32),
        pltpu.VMEM((CHUNK,), jnp.int32),
        pltpu.VMEM((CHUNK + 16,), jnp.int32),
        pltpu.VMEM((CHUNK + 16,), jnp.int32),
        pltpu.VMEM((KB, D), jnp.float32),
        pltpu.VMEM((NODES_PER_W * D,), jnp.float32),
        pltpu.SemaphoreType.DMA,
    ],
  )


def kernel(x, edge_index, batch, norm1_weight, norm1_bias, norm1_mean_scale,
           W_l, b_l, W_r, norm2_weight, norm2_bias, norm2_mean_scale):
    batch2d = batch.reshape(N, 1)
    h1, hwr = _tc_head(x, batch2d, norm1_weight.reshape(1, D),
                       norm1_bias.reshape(1, D),
                       norm1_mean_scale.reshape(1, D), W_r)
    agg_flat = _sc_segmax()(h1, edge_index[0], edge_index[1])
    agg = agg_flat[:N * D].reshape(N, D)
    return _tc_tail(x, agg, hwr, batch2d, W_l, b_l.reshape(1, D),
                    norm2_weight.reshape(1, D), norm2_bias.reshape(1, D),
                    norm2_mean_scale.reshape(1, D))


# speedup vs baseline: 2.9705x; 2.9489x over previous
import functools

import jax
import jax.numpy as jnp
from jax import lax
from jax.experimental import pallas as pl
from jax.experimental.pallas import tpu as pltpu
from jax.experimental.pallas import tpu_sc as plsc

N = 10000
E = 320000
D = 128
G = 8
EPS = 1e-5

NC = 2
NS = 16
NW = NC * NS
NODES_PER_W = 313
NPAD = NW * NODES_PER_W
CHUNK = 4000
NCHUNK = E // CHUNK
GROUPS = CHUNK // 16
KB = 128
NSTAGE = 5120
HKW = D // 2


def _graph_norm_body(x, onehot, counts, weight, bias, mean_scale):
    hi = lax.Precision.HIGHEST
    sums = lax.dot_general(onehot, x, (((0,), (0,)), ((), ())),
                           preferred_element_type=jnp.float32, precision=hi)
    mean = sums / counts
    meanx = lax.dot_general(onehot, mean, (((1,), (0,)), ((), ())),
                            preferred_element_type=jnp.float32, precision=hi)
    out = x - meanx * mean_scale
    var = lax.dot_general(onehot, out * out, (((0,), (0,)), ((), ())),
                          preferred_element_type=jnp.float32,
                          precision=hi) / counts
    stdg = jnp.sqrt(var + EPS)
    stdx = lax.dot_general(onehot, stdg, (((1,), (0,)), ((), ())),
                           preferred_element_type=jnp.float32, precision=hi)
    return weight * out / stdx + bias


def _tc_pack_body(h1_ref, ei_ref, hp_ref, ep_ref):
    h1 = h1_ref[...]
    lo = lax.bitcast_convert_type(h1[:, :HKW].astype(jnp.bfloat16),
                                  jnp.uint16).astype(jnp.int32)
    hb = lax.bitcast_convert_type(h1[:, HKW:].astype(jnp.bfloat16),
                                  jnp.uint16).astype(jnp.int32)
    hp_ref[...] = lo | (hb << 16)
    ei = ei_ref[...]
    ep_ref[...] = (ei[0] << 14) | ei[1]


def _tc_head_body(x_ref, batch_ref, w_ref, b_ref, ms_ref, wr_ref,
                  h1_ref, hwr_ref):
    hi = lax.Precision.HIGHEST
    x = x_ref[...]
    bt = batch_ref[...]
    onehot = (bt == lax.broadcasted_iota(jnp.int32, (1, G), 1)
              ).astype(jnp.float32)
    ones = jnp.ones((N, 1), jnp.float32)
    counts = jnp.maximum(
        lax.dot_general(onehot, ones, (((0,), (0,)), ((), ())),
                        preferred_element_type=jnp.float32, precision=hi),
        1.0)
    h1 = _graph_norm_body(x, onehot, counts, w_ref[...], b_ref[...],
                          ms_ref[...])
    h1_ref[...] = h1
    hwr_ref[...] = lax.dot_general(h1, wr_ref[...], (((1,), (1,)), ((), ())),
                                   preferred_element_type=jnp.float32,
                                   precision=hi)


def _tc_tail_body(x_ref, agg_ref, hwr_ref, batch_ref, wl_ref, bl_ref,
                  w_ref, b_ref, ms_ref, out_ref):
    hi = lax.Precision.HIGHEST
    x = x_ref[...]
    t = lax.dot_general(agg_ref[...], wl_ref[...], (((1,), (1,)), ((), ())),
                        preferred_element_type=jnp.float32, precision=hi)
    h2 = jnp.maximum(x + t + bl_ref[...] + hwr_ref[...], 0.0)
    bt = batch_ref[...]
    onehot = (bt == lax.broadcasted_iota(jnp.int32, (1, G), 1)
              ).astype(jnp.float32)
    ones = jnp.ones((N, 1), jnp.float32)
    counts = jnp.maximum(
        lax.dot_general(onehot, ones, (((0,), (0,)), ((), ())),
                        preferred_element_type=jnp.float32, precision=hi),
        1.0)
    out_ref[...] = _graph_norm_body(h2, onehot, counts, w_ref[...],
                                    b_ref[...], ms_ref[...])


_tc_head = pl.pallas_call(
    _tc_head_body,
    out_shape=(jax.ShapeDtypeStruct((N, D), jnp.float32),
               jax.ShapeDtypeStruct((N, D), jnp.float32)),
)

_tc_pack = pl.pallas_call(
    _tc_pack_body,
    out_shape=(jax.ShapeDtypeStruct((N, HKW), jnp.int32),
               jax.ShapeDtypeStruct((E // 128, 128), jnp.int32)),
)

_tc_tail = pl.pallas_call(
    _tc_tail_body,
    out_shape=jax.ShapeDtypeStruct((N, D), jnp.float32),
)


def _sc_segmax_body(hp_hbm, ep_hbm, out_hbm,
                    ebuf_a, ebuf_b, gat_idx, loc_idx,
                    rows_a, rows_b, agg_v, spm_h,
                    csem_a, csem_b, gsem_a, gsem_b, hsem):
    sid = lax.axis_index("s")
    wid = sid * NC + lax.axis_index("c")
    lo = wid * NODES_PER_W
    iota = lax.iota(jnp.int32, 16)
    ninf = jnp.full((16,), -jnp.inf, jnp.float32)

    HROWS = NSTAGE // NS
    pltpu.async_copy(hp_hbm.at[pl.ds(sid * HROWS, HROWS)],
                     spm_h.at[pl.ds(sid * HROWS, HROWS)], hsem)

    def fire_chunk(c, ebuf, sem):
        pltpu.async_copy(ep_hbm.at[pl.ds(c * CHUNK, CHUNK)], ebuf, sem)

    def wait_chunk(c, ebuf, sem):
        pltpu.make_async_copy(ep_hbm.at[pl.ds(c * CHUNK, CHUNK)], ebuf,
                              sem).wait()

    fire_chunk(0, ebuf_a, csem_a)

    def init_body(i, _):
        idxv = i * 16 + iota
        plsc.store_scatter(agg_v, [idxv], ninf)
        return 0
    lax.fori_loop(0, NODES_PER_W * D // 16, init_body, 0)

    def init_idx(i, _):
        idxv = i * 16 + iota
        plsc.store_scatter(gat_idx, [idxv], jnp.zeros((16,), jnp.int32))
        return 0
    lax.fori_loop(0, (CHUNK + 16) // 16, init_idx, 0)

    pltpu.make_async_copy(hp_hbm.at[pl.ds(sid * HROWS, HROWS)],
                          spm_h.at[pl.ds(sid * HROWS, HROWS)], hsem).wait()
    plsc.subcore_barrier()

    lov = jnp.full((16,), lo, jnp.int32)
    hiv = lov + NODES_PER_W
    dmask = jnp.full((16,), 16383, jnp.int32)
    himask = jnp.full((16,), -65536, jnp.int32)

    def gat_slice(b):
        return spm_h.at[gat_idx.at[pl.ds(b * KB, KB)]]

    def process_batch(b, n, rows_v):
        k = jnp.minimum(n - b * KB, KB)

        def edge_body(j, _):
            lvv = plsc.load_gather(loc_idx, [jnp.full((16,), b * KB + j,
                                                      jnp.int32)])
            dv = lvv & jnp.full((16,), 511, jnp.int32)
            par = lax.shift_right_logical(lvv, 9)
            base = dv * D
            rj = jnp.full((16,), j, jnp.int32)
            cols = [iota + c4 * 16 for c4 in range(HKW // 16)]
            rcols = [cv + par * HKW for cv in cols]
            vis = [plsc.load_gather(rows_v, [rj, cv]) for cv in rcols]
            a_lo = [plsc.load_gather(agg_v, [base + cv]) for cv in cols]
            a_hi = [plsc.load_gather(agg_v, [base + HKW + cv]) for cv in cols]
            r_lo = [plsc.bitcast(v << 16, jnp.float32) for v in vis]
            r_hi = [plsc.bitcast(v & himask, jnp.float32) for v in vis]
            m_lo = [jnp.maximum(a, r) for a, r in zip(a_lo, r_lo)]
            m_hi = [jnp.maximum(a, r) for a, r in zip(a_hi, r_hi)]
            for cv, v in zip(cols, m_lo):
                plsc.store_scatter(agg_v, [base + cv], v)
            for cv, v in zip(cols, m_hi):
                plsc.store_scatter(agg_v, [base + HKW + cv], v)
            return 0

        lax.fori_loop(0, k, edge_body, 0)

    def process_chunk(ebuf):
        def scan_body(g, n_vec):
            ev = g * 16 + iota
            p16 = plsc.load_gather(ebuf, [ev])
            d16 = p16 & dmask
            s16 = lax.shift_right_logical(p16, 14)
            m = (d16 >= lov) & (d16 < hiv)
            pref = plsc.cumsum(jnp.where(m, 1, 0))
            pos = n_vec + pref - 1
            srow = lax.shift_right_logical(s16, 1)
            lv = (d16 - lov) | ((s16 & jnp.full((16,), 1, jnp.int32)) << 9)
            plsc.store_scatter(gat_idx, [pos], srow, mask=m)
            plsc.store_scatter(loc_idx, [pos], lv, mask=m)
            return n_vec + plsc.all_reduce_population_count(m)

        n_vec = lax.fori_loop(0, GROUPS, scan_body,
                              jnp.zeros((16,), jnp.int32))
        n = jnp.max(n_vec)

        nb = (n + KB - 1) // KB

        @pl.when(nb > 0)
        def _():
            pltpu.async_copy(gat_slice(0), rows_a, gsem_a)

        def gpair_body(q, _):
            b0 = 2 * q
            b1 = b0 + 1

            @pl.when(b1 < nb)
            def _():
                pltpu.async_copy(gat_slice(b1), rows_b, gsem_b)
            pltpu.make_async_copy(gat_slice(b0), rows_a, gsem_a).wait()
            process_batch(b0, n, rows_a)

            @pl.when(b1 < nb)
            def _():
                @pl.when(b0 + 2 < nb)
                def _():
                    pltpu.async_copy(gat_slice(b0 + 2), rows_a, gsem_a)
                pltpu.make_async_copy(gat_slice(b1), rows_b, gsem_b).wait()
                process_batch(b1, n, rows_b)
            return 0

        lax.fori_loop(0, (nb + 1) // 2, gpair_body, 0)

    def pair_body(p, _):
        c0 = 2 * p
        c1 = c0 + 1
        fire_chunk(c1, ebuf_b, csem_b)
        wait_chunk(c0, ebuf_a, csem_a)
        process_chunk(ebuf_a)

        @pl.when(c0 + 2 < NCHUNK)
        def _():
            fire_chunk(c0 + 2, ebuf_a, csem_a)
        wait_chunk(c1, ebuf_b, csem_b)
        process_chunk(ebuf_b)
        return 0

    lax.fori_loop(0, NCHUNK // 2, pair_body, 0)

    def fix_body(i, _):
        idxv = i * 16 + iota
        v = plsc.load_gather(agg_v, [idxv])
        v = jnp.where(v == ninf, 0.0, v)
        plsc.store_scatter(agg_v, [idxv], v)
        return 0
    lax.fori_loop(0, NODES_PER_W * D // 16, fix_body, 0)

    pltpu.sync_copy(agg_v, out_hbm.at[pl.ds(lo * D, NODES_PER_W * D)])


@functools.cache
def _sc_segmax():
  return pl.kernel(
    _sc_segmax_body,
    out_type=jax.ShapeDtypeStruct((NPAD * D,), jnp.float32),
    mesh=plsc.VectorSubcoreMesh(core_axis_name="c", subcore_axis_name="s",
                                num_cores=NC, num_subcores=NS),
    compiler_params=pltpu.CompilerParams(needs_layout_passes=False),
    scratch_types=[
        pltpu.VMEM((CHUNK,), jnp.int32),
        pltpu.VMEM((CHUNK,), jnp.int32),
        pltpu.VMEM((CHUNK + 16,), jnp.int32),
        pltpu.VMEM((CHUNK + 16,), jnp.int32),
        pltpu.VMEM((KB, D), jnp.int32),
        pltpu.VMEM((KB, D), jnp.int32),
        pltpu.VMEM((NODES_PER_W * D,), jnp.float32),
        pltpu.VMEM_SHARED((NSTAGE, D), jnp.int32),
        pltpu.SemaphoreType.DMA,
        pltpu.SemaphoreType.DMA,
        pltpu.SemaphoreType.DMA,
        pltpu.SemaphoreType.DMA,
        pltpu.SemaphoreType.DMA,
    ],
  )


def kernel(x, edge_index, batch, norm1_weight, norm1_bias, norm1_mean_scale,
           W_l, b_l, W_r, norm2_weight, norm2_bias, norm2_mean_scale):
    batch2d = batch.reshape(N, 1)
    h1, hwr = _tc_head(x, batch2d, norm1_weight.reshape(1, D),
                       norm1_bias.reshape(1, D),
                       norm1_mean_scale.reshape(1, D), W_r)
    hp, ep = _tc_pack(h1, edge_index.reshape(2, E // 128, 128))
    hp2 = hp.reshape(N // 2, D)
    hp_pad = jnp.pad(hp2, ((0, NSTAGE - N // 2), (0, 0)))
    agg_flat = _sc_segmax()(hp_pad, ep.reshape(E), )
    agg = agg_flat[:N * D].reshape(N, D)
    return _tc_tail(x, agg, hwr, batch2d, W_l, b_l.reshape(1, D),
                    norm2_weight.reshape(1, D), norm2_bias.reshape(1, D),
                    norm2_mean_scale.reshape(1, D))
